# baseline (speedup 1.0000x reference)
import jax
import jax.numpy as jnp
import numpy as np
from jax.experimental import pallas as pl
from jax.experimental.pallas import tpu as pltpu

B, C, H, W = 4, 96, 224, 224
OC = 96
DEPTH = 3
NUM_LEAVES = 2 ** DEPTH
NUM_NODES = NUM_LEAVES - 1
HO, WO = 112, 112
HC = 232
NCHUNK = 4
CCH = C // NCHUNK

_GATE_TABLE = np.array([
    [0, 0, 0, 0],
    [0, 0, 0, 1],
    [0, 1, 0, -1],
    [0, 1, 0, 0],
    [0, 0, 1, -1],
    [0, 0, 1, 0],
    [0, 1, 1, -2],
    [0, 1, 1, -1],
    [1, -1, -1, 1],
    [1, -1, -1, 2],
    [1, 0, -1, 0],
    [1, 0, -1, 1],
    [1, -1, 0, 0],
    [1, -1, 0, 1],
    [1, 0, 0, -1],
    [1, 0, 0, 0],
], dtype=np.float32)


def _tree_kernel(lid_ref, coef_ref, x_ref, out_ref, xc_ref):
    b = pl.program_id(0)
    k = pl.program_id(1)

    @pl.when((b == 0) & (k == 0))
    def _zero_pad_rows():
        xc_ref[:, :, 0:8, :] = jnp.zeros((3, C, 8, 128), jnp.float32)

    rows_i = jax.lax.broadcasted_iota(jnp.int32, (W, 384), 0)
    cols = jax.lax.broadcasted_iota(jnp.int32, (W, 384), 1)
    sel = (rows_i == 2 * (cols & 127) + (cols >> 7) - 1).astype(jnp.bfloat16)

    def chan_body(cc, carry):
        for u in range(24):
            v = x_ref[0, 24 * cc + u]
            cg = k * CCH + 24 * cc + u
            v_hi = v.astype(jnp.bfloat16)
            v_lo = (v - v_hi.astype(jnp.float32)).astype(jnp.bfloat16)
            m = (jax.lax.dot(v_hi, sel, preferred_element_type=jnp.float32)
                 + jax.lax.dot(v_lo, sel, preferred_element_type=jnp.float32))
            xc_ref[0, cg, 8:HC, :] = m[:, 0:128]
            xc_ref[1, cg, 8:HC, :] = m[:, 128:256]
            xc_ref[2, cg, 8:HC, :] = m[:, 256:384]
        return carry

    jax.lax.fori_loop(0, CCH // 24, chan_body, 0)

    @pl.when(k == NCHUNK - 1)
    def _tree():
        def oc_body(oc2, carry):
            for u in range(16):
                oc = 16 * oc2 + u
                cur = []
                for l in range(NUM_LEAVES):
                    c = lid_ref[0, oc, l]
                    dy = lid_ref[1, oc, l]
                    dx = lid_ref[2, oc, l]
                    leaf = xc_ref[dx, c, pl.Slice(dy + 7, HO, 2), :]
                    cur.append(leaf)
                node = 0
                for _level in range(DEPTH):
                    nxt = []
                    for i in range(len(cur) // 2):
                        a = cur[2 * i]
                        b2 = cur[2 * i + 1]
                        k0 = coef_ref[oc, node, 0]
                        ka = coef_ref[oc, node, 1]
                        kb = coef_ref[oc, node, 2]
                        kab = coef_ref[oc, node, 3]
                        nxt.append(k0 + ka * a + kb * b2 + kab * (a * b2))
                        node += 1
                    cur = nxt
                out_ref[0, oc] = cur[0][:, 0:WO]
            return carry

        jax.lax.fori_loop(0, OC // 16, oc_body, 0)


def kernel(x, logits, leaf_indices):
    probs = jax.nn.softmax(logits, axis=-1)
    coef = jnp.einsum('onk,kj->onj', probs, jnp.asarray(_GATE_TABLE))
    lid = jnp.stack([leaf_indices // 9,
                     (leaf_indices % 9) // 3,
                     leaf_indices % 3]).astype(jnp.int32)
    out = pl.pallas_call(
        _tree_kernel,
        grid=(B, NCHUNK),
        in_specs=[
            pl.BlockSpec(memory_space=pltpu.SMEM),
            pl.BlockSpec(memory_space=pltpu.SMEM),
            pl.BlockSpec((1, CCH, H, W), lambda b, k: (b, k, 0, 0)),
        ],
        out_specs=pl.BlockSpec((1, OC, HO, WO), lambda b, k: (b, 0, 0, 0)),
        out_shape=jax.ShapeDtypeStruct((B, OC, HO, WO), jnp.float32),
        scratch_shapes=[pltpu.VMEM((3, C, HC, 128), jnp.float32)],
        compiler_params=pltpu.CompilerParams(
            dimension_semantics=("arbitrary", "arbitrary"),
            vmem_limit_bytes=62 * 1024 * 1024,
        ),
    )(lid, coef, x)
    return out


# speedup vs baseline: 1.1542x; 1.1542x over previous
import jax
import jax.numpy as jnp
import numpy as np
from jax.experimental import pallas as pl
from jax.experimental.pallas import tpu as pltpu

B, C, H, W = 4, 96, 224, 224
OC = 96
DEPTH = 3
NUM_LEAVES = 2 ** DEPTH
NUM_NODES = NUM_LEAVES - 1
HO, WO = 112, 112
HC = 232
NCHUNK = 4
CCH = C // NCHUNK

_GATE_TABLE = np.array([
    [0, 0, 0, 0],
    [0, 0, 0, 1],
    [0, 1, 0, -1],
    [0, 1, 0, 0],
    [0, 0, 1, -1],
    [0, 0, 1, 0],
    [0, 1, 1, -2],
    [0, 1, 1, -1],
    [1, -1, -1, 1],
    [1, -1, -1, 2],
    [1, 0, -1, 0],
    [1, 0, -1, 1],
    [1, -1, 0, 0],
    [1, -1, 0, 1],
    [1, 0, 0, -1],
    [1, 0, 0, 0],
], dtype=np.float32)


def _tree_kernel(lid_ref, coef_ref, x_ref, out_ref, xc_ref):
    b = pl.program_id(0)
    k = pl.program_id(1)

    @pl.when((b == 0) & (k == 0))
    def _zero_pad_rows():
        xc_ref[:, :, 0:8, :] = jnp.zeros((3, C, 8, 128), jnp.float32)

    rows_i = jax.lax.broadcasted_iota(jnp.int32, (W, 256), 0)
    cols = jax.lax.broadcasted_iota(jnp.int32, (W, 256), 1)
    sel = (rows_i == 2 * (cols & 127) + (cols >> 7)).astype(jnp.bfloat16)

    def chan_body(cc, carry):
        for u in range(24):
            v = x_ref[0, 24 * cc + u]
            cg = k * CCH + 24 * cc + u
            v_hi = v.astype(jnp.bfloat16)
            v_lo = (v - v_hi.astype(jnp.float32)).astype(jnp.bfloat16)
            m = (jax.lax.dot(v_hi, sel, preferred_element_type=jnp.float32)
                 + jax.lax.dot(v_lo, sel, preferred_element_type=jnp.float32))
            ev = m[:, 0:128]
            od = m[:, 128:256]
            xc_ref[1, cg, 8:HC, :] = ev
            xc_ref[2, cg, 8:HC, :] = od
            xc_ref[0, cg, 8:HC, :] = jnp.concatenate(
                [jnp.zeros((H, 1), jnp.float32), od[:, 0:127]], axis=1)
        return carry

    jax.lax.fori_loop(0, CCH // 24, chan_body, 0)

    @pl.when(k == NCHUNK - 1)
    def _tree():
        def oc_body(oc2, carry):
            for u in range(16):
                oc = 16 * oc2 + u
                cur = []
                for l in range(NUM_LEAVES):
                    c = lid_ref[0, oc, l]
                    dy = lid_ref[1, oc, l]
                    dx = lid_ref[2, oc, l]
                    leaf = xc_ref[dx, c, pl.Slice(dy + 7, HO, 2), :]
                    cur.append(leaf)
                node = 0
                for _level in range(DEPTH):
                    nxt = []
                    for i in range(len(cur) // 2):
                        a = cur[2 * i]
                        b2 = cur[2 * i + 1]
                        k0 = coef_ref[oc, node, 0]
                        ka = coef_ref[oc, node, 1]
                        kb = coef_ref[oc, node, 2]
                        kab = coef_ref[oc, node, 3]
                        nxt.append(k0 + ka * a + kb * b2 + kab * (a * b2))
                        node += 1
                    cur = nxt
                out_ref[0, oc] = cur[0][:, 0:WO]
            return carry

        jax.lax.fori_loop(0, OC // 16, oc_body, 0)


def kernel(x, logits, leaf_indices):
    probs = jax.nn.softmax(logits, axis=-1)
    coef = jnp.einsum('onk,kj->onj', probs, jnp.asarray(_GATE_TABLE))
    lid = jnp.stack([leaf_indices // 9,
                     (leaf_indices % 9) // 3,
                     leaf_indices % 3]).astype(jnp.int32)
    out = pl.pallas_call(
        _tree_kernel,
        grid=(B, NCHUNK),
        in_specs=[
            pl.BlockSpec(memory_space=pltpu.SMEM),
            pl.BlockSpec(memory_space=pltpu.SMEM),
            pl.BlockSpec((1, CCH, H, W), lambda b, k: (b, k, 0, 0)),
        ],
        out_specs=pl.BlockSpec((1, OC, HO, WO), lambda b, k: (b, 0, 0, 0)),
        out_shape=jax.ShapeDtypeStruct((B, OC, HO, WO), jnp.float32),
        scratch_shapes=[pltpu.VMEM((3, C, HC, 128), jnp.float32)],
        compiler_params=pltpu.CompilerParams(
            dimension_semantics=("arbitrary", "arbitrary"),
            vmem_limit_bytes=62 * 1024 * 1024,
        ),
    )(lid, coef, x)
    return out
